# baseline (speedup 1.0000x reference)
import functools

import jax
import jax.numpy as jnp
from jax import lax
from jax.experimental import pallas as pl
from jax.experimental.pallas import tpu as pltpu
from jax.experimental.pallas import tpu_sc as plsc

_CHUNK = 128


def _gather_sc(table, idx):
    info = plsc.get_sparse_core_info()
    nc, ns = info.num_cores, info.num_subcores
    nw = nc * ns
    b = idx.shape[0]
    d = table.shape[1]
    b_per_w = b // nw
    nchunk = b_per_w // _CHUNK
    mesh = plsc.VectorSubcoreMesh(core_axis_name="c", subcore_axis_name="s")

    @functools.partial(
        pl.kernel,
        mesh=mesh,
        out_type=jax.ShapeDtypeStruct((b, d), jnp.float32),
        scratch_types=[
            pltpu.VMEM((b_per_w,), jnp.int32),
            pltpu.VMEM((b_per_w, d), jnp.float32),
            pltpu.SemaphoreType.DMA,
        ],
        compiler_params=pltpu.CompilerParams(use_tc_tiling_on_sc=False),
    )
    def k(table_hbm, idx_hbm, out_hbm, idx_v, rows_v, sem):
        wid = lax.axis_index("s") * nc + lax.axis_index("c")
        base = wid * b_per_w
        pltpu.sync_copy(idx_hbm.at[pl.ds(base, b_per_w)], idx_v)
        copies = []
        for j in range(nchunk):
            copies.append(
                pltpu.async_copy(
                    table_hbm.at[idx_v.at[pl.ds(j * _CHUNK, _CHUNK)]],
                    rows_v.at[pl.ds(j * _CHUNK, _CHUNK)],
                    sem,
                )
            )
        for c in copies:
            c.wait()
        pltpu.sync_copy(rows_v, out_hbm.at[pl.ds(base, b_per_w)])

    return k(table, idx)


def _mm_body(e_ref, w_ref, b_ref, o_ref):
    o_ref[...] = (
        jnp.dot(e_ref[...], w_ref[...], preferred_element_type=jnp.float32)
        + b_ref[...]
    )


def _linear_tc(emb, wt, b2):
    bsz, d = emb.shape
    c = wt.shape[1]
    bm = 2048
    return pl.pallas_call(
        _mm_body,
        grid=(bsz // bm,),
        in_specs=[
            pl.BlockSpec((bm, d), lambda i: (i, 0)),
            pl.BlockSpec((d, c), lambda i: (0, 0)),
            pl.BlockSpec((1, c), lambda i: (0, 0)),
        ],
        out_specs=pl.BlockSpec((bm, c), lambda i: (i, 0)),
        out_shape=jax.ShapeDtypeStruct((bsz, c), jnp.float32),
    )(emb, wt, b2)


def kernel(word_idx, table, W, b):
    emb = _gather_sc(table, word_idx.astype(jnp.int32))
    return _linear_tc(emb, W.T, b.reshape(1, -1))


# speedup vs baseline: 1.6315x; 1.6315x over previous
import functools

import jax
import jax.numpy as jnp
from jax import lax
from jax.experimental import pallas as pl
from jax.experimental.pallas import tpu as pltpu
from jax.experimental.pallas import tpu_sc as plsc


def _gather_sc(table, idx):
    nc = 2
    ns = 16
    nw = nc * ns
    b = idx.shape[0]
    d = table.shape[1]
    b_per_w = b // nw
    mesh = plsc.VectorSubcoreMesh(core_axis_name="c", subcore_axis_name="s")

    @functools.partial(
        pl.kernel,
        mesh=mesh,
        out_type=jax.ShapeDtypeStruct((b, d), jnp.float32),
        scratch_types=[
            pltpu.VMEM((b_per_w,), jnp.int32),
            pltpu.VMEM((b_per_w, d), jnp.float32),
            pltpu.SemaphoreType.DMA,
        ],
        compiler_params=pltpu.CompilerParams(needs_layout_passes=False),
    )
    def k(table_hbm, idx_hbm, out_hbm, idx_v, rows_v, sem):
        wid = lax.axis_index("s") * nc + lax.axis_index("c")
        base = wid * b_per_w
        pltpu.sync_copy(idx_hbm.at[pl.ds(base, b_per_w)], idx_v)
        lanes = lax.iota(jnp.int32, 16)

        def fire(g, _):
            vec = idx_v[pl.ds(g * 16, 16)]
            for l in range(16):
                row = jnp.sum(jnp.where(lanes == l, vec, 0))
                pltpu.make_async_copy(
                    table_hbm.at[pl.ds(row, 1)],
                    rows_v.at[pl.ds(g * 16 + l, 1)],
                    sem,
                ).start()
            return _

        lax.fori_loop(0, b_per_w // 16, fire, 0)

        def drain(i, _):
            pltpu.make_async_copy(
                table_hbm.at[pl.ds(0, 1)], rows_v.at[pl.ds(i, 1)], sem
            ).wait()
            return _

        lax.fori_loop(0, b_per_w, drain, 0)
        pltpu.sync_copy(rows_v, out_hbm.at[pl.ds(base, b_per_w)])

    return k(table, idx)


def _mm_body(e_ref, w_ref, b_ref, o_ref):
    o_ref[...] = (
        jnp.dot(e_ref[...], w_ref[...], preferred_element_type=jnp.float32)
        + b_ref[...]
    )


def _linear_tc(emb, wt, b2):
    bsz, d = emb.shape
    c = wt.shape[1]
    bm = 2048
    return pl.pallas_call(
        _mm_body,
        grid=(bsz // bm,),
        in_specs=[
            pl.BlockSpec((bm, d), lambda i: (i, 0)),
            pl.BlockSpec((d, c), lambda i: (0, 0)),
            pl.BlockSpec((1, c), lambda i: (0, 0)),
        ],
        out_specs=pl.BlockSpec((bm, c), lambda i: (i, 0)),
        out_shape=jax.ShapeDtypeStruct((bsz, c), jnp.float32),
    )(emb, wt, b2)


def kernel(word_idx, table, W, b):
    emb = _gather_sc(table, word_idx.astype(jnp.int32))
    return _linear_tc(emb, W.T, b.reshape(1, -1))


# speedup vs baseline: 3.8304x; 2.3478x over previous
import functools

import jax
import jax.numpy as jnp
from jax import lax
from jax.experimental import pallas as pl
from jax.experimental.pallas import tpu as pltpu
from jax.experimental.pallas import tpu_sc as plsc

_NBUF = 8


def _gather_sc(tablet, idx):
    nc = 2
    ns = 16
    nw = nc * ns
    b = idx.shape[0]
    d = tablet.shape[0]
    b_per_w = b // nw
    mesh = plsc.VectorSubcoreMesh(core_axis_name="c", subcore_axis_name="s")

    @functools.partial(
        pl.kernel,
        mesh=mesh,
        out_type=jax.ShapeDtypeStruct((d, b), jnp.float32),
        scratch_types=[
            pltpu.VMEM((b_per_w,), jnp.int32),
            pltpu.VMEM((_NBUF, d, 128), jnp.float32),
            pltpu.VMEM((d, b_per_w), jnp.float32),
            pltpu.SemaphoreType.DMA((_NBUF,)),
        ],
        compiler_params=pltpu.CompilerParams(needs_layout_passes=False),
    )
    def k(tablet_hbm, idx_hbm, out_hbm, idx_v, blk_v, embt_v, sems):
        wid = lax.axis_index("s") * nc + lax.axis_index("c")
        base = wid * b_per_w
        pltpu.sync_copy(idx_hbm.at[pl.ds(base, b_per_w)], idx_v)
        lanes = lax.iota(jnp.int32, 16)

        def row_of(j):
            vec = idx_v[pl.ds((j >> 4) * 16, 16)]
            return jnp.sum(jnp.where(lanes == (j & 15), vec, 0))

        def fire(j, slot):
            row = row_of(j)
            cb = pl.multiple_of((row >> 7) * 128, 128)
            pltpu.make_async_copy(
                tablet_hbm.at[:, pl.ds(cb, 128)], blk_v.at[slot], sems.at[slot]
            ).start()

        def extract(j, slot):
            row = row_of(j)
            lane = jnp.full((16,), row & 127, jnp.int32)
            col = jnp.full((16,), j, jnp.int32)
            for h in range(d // 16):
                part = plsc.load_gather(blk_v.at[slot], [lanes + h * 16, lane])
                plsc.store_scatter(embt_v, [lanes + h * 16, col], part)

        for s in range(_NBUF):
            fire(jnp.int32(s), s)

        def round_body(r, carry):
            for s in range(_NBUF):
                j = r * _NBUF + s
                pltpu.make_async_copy(
                    tablet_hbm.at[:, pl.ds(0, 128)], blk_v.at[s], sems.at[s]
                ).wait()
                extract(j, s)

                @pl.when(r < b_per_w // _NBUF - 1)
                def _refire(j=j, s=s):
                    fire(j + _NBUF, s)

            return carry

        lax.fori_loop(0, b_per_w // _NBUF, round_body, 0)
        pltpu.sync_copy(embt_v, out_hbm.at[:, pl.ds(base, b_per_w)])

    return k(tablet, idx)


def _mm_body(e_ref, w_ref, b_ref, o_ref):
    o_ref[...] = (
        lax.dot_general(
            e_ref[...],
            w_ref[...],
            dimension_numbers=(((0,), (0,)), ((), ())),
            preferred_element_type=jnp.float32,
        )
        + b_ref[...]
    )


def _linear_tc(embt, wt, b2):
    d, bsz = embt.shape
    c = wt.shape[1]
    bm = 2048
    return pl.pallas_call(
        _mm_body,
        grid=(bsz // bm,),
        in_specs=[
            pl.BlockSpec((d, bm), lambda i: (0, i)),
            pl.BlockSpec((d, c), lambda i: (0, 0)),
            pl.BlockSpec((1, c), lambda i: (0, 0)),
        ],
        out_specs=pl.BlockSpec((bm, c), lambda i: (i, 0)),
        out_shape=jax.ShapeDtypeStruct((bsz, c), jnp.float32),
    )(embt, wt, b2)


def kernel(word_idx, table, W, b):
    embt = _gather_sc(table.T, word_idx.astype(jnp.int32))
    return _linear_tc(embt, W.T, b.reshape(1, -1))


# speedup vs baseline: 3.9178x; 1.0228x over previous
import functools

import jax
import jax.numpy as jnp
from jax import lax
from jax.experimental import pallas as pl
from jax.experimental.pallas import tpu as pltpu
from jax.experimental.pallas import tpu_sc as plsc

_NBUF = 16


def _gather_sc(tablet, idx):
    nc = 2
    ns = 16
    nw = nc * ns
    b = idx.shape[0]
    d = tablet.shape[0]
    b_per_w = b // nw
    n_rounds = b_per_w // _NBUF
    mesh = plsc.VectorSubcoreMesh(core_axis_name="c", subcore_axis_name="s")

    @functools.partial(
        pl.kernel,
        mesh=mesh,
        out_type=jax.ShapeDtypeStruct((d, b), jnp.float32),
        scratch_types=[
            pltpu.VMEM((b_per_w,), jnp.int32),
            pltpu.SMEM((b_per_w,), jnp.int32),
            pltpu.VMEM((_NBUF, d, 128), jnp.float32),
            pltpu.VMEM((d, b_per_w), jnp.float32),
            pltpu.SemaphoreType.DMA((_NBUF,)),
        ],
        compiler_params=pltpu.CompilerParams(needs_layout_passes=False),
    )
    def k(tablet_hbm, idx_hbm, out_hbm, idx_v, idx_s, blk_v, embt_v, sems):
        wid = lax.axis_index("s") * nc + lax.axis_index("c")
        base = wid * b_per_w
        pltpu.sync_copy(idx_hbm.at[pl.ds(base, b_per_w)], idx_v)
        lanes = lax.iota(jnp.int32, 16)

        def stage(g, carry):
            vec = idx_v[pl.ds(g * 16, 16)]
            for l in range(16):
                idx_s[g * 16 + l] = jnp.sum(jnp.where(lanes == l, vec, 0))
            return carry

        lax.fori_loop(0, b_per_w // 16, stage, 0)

        def fire(j, slot):
            row = idx_s[j]
            cb = pl.multiple_of((row >> 7) * 128, 128)
            pltpu.make_async_copy(
                tablet_hbm.at[:, pl.ds(cb, 128)], blk_v.at[slot], sems.at[slot]
            ).start()

        def wait(slot):
            pltpu.make_async_copy(
                tablet_hbm.at[:, pl.ds(0, 128)], blk_v.at[slot], sems.at[slot]
            ).wait()

        def extract(j, slot):
            row = idx_s[j]
            lane = jnp.full((16,), row & 127, jnp.int32)
            col = jnp.full((16,), j, jnp.int32)
            for h in range(d // 16):
                part = plsc.load_gather(blk_v.at[slot], [lanes + h * 16, lane])
                plsc.store_scatter(embt_v, [lanes + h * 16, col], part)

        for s in range(_NBUF):
            fire(jnp.int32(s), s)

        def round_body(r, carry):
            for s in range(_NBUF):
                j = r * _NBUF + s
                wait(s)
                extract(j, s)
                fire(j + _NBUF, s)
            return carry

        lax.fori_loop(0, n_rounds - 1, round_body, 0)
        for s in range(_NBUF):
            j = (n_rounds - 1) * _NBUF + s
            wait(s)
            extract(jnp.int32(j), s)
        pltpu.sync_copy(embt_v, out_hbm.at[:, pl.ds(base, b_per_w)])

    return k(tablet, idx)


def _mm_body(w_ref, e_ref, b_ref, o_ref):
    o_ref[...] = (
        jnp.dot(w_ref[...], e_ref[...], preferred_element_type=jnp.float32)
        + b_ref[...]
    )


def _linear_tc(embt, W, b2):
    d, bsz = embt.shape
    c = W.shape[0]
    bm = 2048
    return pl.pallas_call(
        _mm_body,
        grid=(bsz // bm,),
        in_specs=[
            pl.BlockSpec((c, d), lambda i: (0, 0)),
            pl.BlockSpec((d, bm), lambda i: (0, i)),
            pl.BlockSpec((c, 1), lambda i: (0, 0)),
        ],
        out_specs=pl.BlockSpec((c, bm), lambda i: (0, i)),
        out_shape=jax.ShapeDtypeStruct((c, bsz), jnp.float32),
    )(W, embt, b2)


def kernel(word_idx, table, W, b):
    embt = _gather_sc(table.T, word_idx.astype(jnp.int32))
    out_t = _linear_tc(embt, W, b.reshape(-1, 1))
    return out_t.T


# speedup vs baseline: 3.9750x; 1.0146x over previous
import functools

import jax
import jax.numpy as jnp
from jax import lax
from jax.experimental import pallas as pl
from jax.experimental.pallas import tpu as pltpu
from jax.experimental.pallas import tpu_sc as plsc

_NBUF = 16


def _gather_sc(tablet, idx):
    nc = 2
    ns = 16
    nw = nc * ns
    b = idx.shape[0]
    d = tablet.shape[0]
    b_per_w = b // nw
    n_rounds = b_per_w // _NBUF
    mesh = plsc.VectorSubcoreMesh(core_axis_name="c", subcore_axis_name="s")

    @functools.partial(
        pl.kernel,
        mesh=mesh,
        out_type=jax.ShapeDtypeStruct((d, b), jnp.float32),
        scratch_types=[
            pltpu.VMEM((b_per_w,), jnp.int32),
            pltpu.SMEM((b_per_w,), jnp.int32),
            pltpu.VMEM((_NBUF, d, 128), jnp.float32),
            pltpu.VMEM((d, b_per_w), jnp.float32),
            pltpu.SemaphoreType.DMA((_NBUF,)),
        ],
        compiler_params=pltpu.CompilerParams(needs_layout_passes=False),
    )
    def k(tablet_hbm, idx_hbm, out_hbm, idx_v, idx_s, blk_v, embt_v, sems):
        wid = lax.axis_index("s") * nc + lax.axis_index("c")
        base = wid * b_per_w
        pltpu.sync_copy(idx_hbm.at[pl.ds(base, b_per_w)], idx_v)
        lanes = lax.iota(jnp.int32, 16)

        def stage(g, carry):
            vec = idx_v[pl.ds(g * 16, 16)]
            for l in range(16):
                idx_s[g * 16 + l] = jnp.sum(jnp.where(lanes == l, vec, 0))
            return carry

        lax.fori_loop(0, b_per_w // 16, stage, 0)

        def fire(j, slot):
            row = idx_s[j]
            cb = pl.multiple_of((row >> 7) * 128, 128)
            for g4 in range(d // 8):
                pltpu.make_async_copy(
                    tablet_hbm.at[pl.ds(g4 * 8, 8), pl.ds(cb, 128)],
                    blk_v.at[slot, pl.ds(g4 * 8, 8)],
                    sems.at[slot],
                ).start()

        def wait(slot):
            pltpu.make_async_copy(
                tablet_hbm.at[:, pl.ds(0, 128)], blk_v.at[slot], sems.at[slot]
            ).wait()

        def extract(j, slot):
            row = idx_s[j]
            lane = jnp.full((16,), row & 127, jnp.int32)
            col = jnp.full((16,), j, jnp.int32)
            for h in range(d // 16):
                part = plsc.load_gather(blk_v.at[slot], [lanes + h * 16, lane])
                plsc.store_scatter(embt_v, [lanes + h * 16, col], part)

        for s in range(_NBUF):
            fire(jnp.int32(s), s)

        def round_body(r, carry):
            for s in range(_NBUF):
                j = r * _NBUF + s
                wait(s)
                extract(j, s)
                fire(j + _NBUF, s)
            return carry

        lax.fori_loop(0, n_rounds - 1, round_body, 0)
        for s in range(_NBUF):
            j = (n_rounds - 1) * _NBUF + s
            wait(s)
            extract(jnp.int32(j), s)
        pltpu.sync_copy(embt_v, out_hbm.at[:, pl.ds(base, b_per_w)])

    return k(tablet, idx)


def _mm_body(w_ref, e_ref, b_ref, o_ref):
    o_ref[...] = (
        jnp.dot(w_ref[...], e_ref[...], preferred_element_type=jnp.float32)
        + b_ref[...]
    )


def _linear_tc(embt, W, b2):
    d, bsz = embt.shape
    c = W.shape[0]
    bm = 2048
    return pl.pallas_call(
        _mm_body,
        grid=(bsz // bm,),
        in_specs=[
            pl.BlockSpec((c, d), lambda i: (0, 0)),
            pl.BlockSpec((d, bm), lambda i: (0, i)),
            pl.BlockSpec((c, 1), lambda i: (0, 0)),
        ],
        out_specs=pl.BlockSpec((c, bm), lambda i: (0, i)),
        out_shape=jax.ShapeDtypeStruct((c, bsz), jnp.float32),
    )(W, embt, b2)


def kernel(word_idx, table, W, b):
    embt = _gather_sc(table.T, word_idx.astype(jnp.int32))
    out_t = _linear_tc(embt, W, b.reshape(-1, 1))
    return out_t.T


# speedup vs baseline: 4.0259x; 1.0128x over previous
import functools

import jax
import jax.numpy as jnp
from jax import lax
from jax.experimental import pallas as pl
from jax.experimental.pallas import tpu as pltpu
from jax.experimental.pallas import tpu_sc as plsc

_NBUF = 16


def _gather_sc(tablet, idx):
    nc = 2
    ns = 16
    nw = nc * ns
    b = idx.shape[0]
    d = tablet.shape[0]
    b_per_w = b // nw
    n_rounds = b_per_w // _NBUF
    mesh = plsc.VectorSubcoreMesh(core_axis_name="c", subcore_axis_name="s")

    @functools.partial(
        pl.kernel,
        mesh=mesh,
        out_type=jax.ShapeDtypeStruct((d, b), jnp.float32),
        scratch_types=[
            pltpu.VMEM((b_per_w,), jnp.int32),
            pltpu.SMEM((b_per_w,), jnp.int32),
            pltpu.VMEM((_NBUF, d, 128), jnp.float32),
            pltpu.VMEM((d, b_per_w), jnp.float32),
            pltpu.SemaphoreType.DMA((_NBUF,)),
        ],
        compiler_params=pltpu.CompilerParams(needs_layout_passes=False),
    )
    def k(tablet_hbm, idx_hbm, out_hbm, idx_v, idx_s, blk_v, embt_v, sems):
        wid = lax.axis_index("s") * nc + lax.axis_index("c")
        base = wid * b_per_w
        pltpu.sync_copy(idx_hbm.at[pl.ds(base, b_per_w)], idx_v)
        lanes = lax.iota(jnp.int32, 16)

        def stage(g, carry):
            vec = idx_v[pl.ds(g * 16, 16)]
            for l in range(16):
                idx_s[g * 16 + l] = jnp.sum(jnp.where(lanes == l, vec, 0))
            return carry

        lax.fori_loop(0, b_per_w // 16, stage, 0)

        def fire(j, slot):
            row = idx_s[j]
            cb = pl.multiple_of((row >> 7) * 128, 128)
            for g4 in range(d // 8):
                pltpu.make_async_copy(
                    tablet_hbm.at[pl.ds(g4 * 8, 8), pl.ds(cb, 128)],
                    blk_v.at[slot, pl.ds(g4 * 8, 8)],
                    sems.at[slot],
                ).start()

        def wait(slot):
            pltpu.make_async_copy(
                tablet_hbm.at[:, pl.ds(0, 128)], blk_v.at[slot], sems.at[slot]
            ).wait()

        def extract(j, slot):
            row = idx_s[j]
            lane = jnp.full((16,), row & 127, jnp.int32)
            col = jnp.full((16,), j, jnp.int32)
            for h in range(d // 16):
                part = plsc.load_gather(blk_v.at[slot], [lanes + h * 16, lane])
                plsc.store_scatter(embt_v, [lanes + h * 16, col], part)

        for s in range(_NBUF):
            fire(jnp.int32(s), s)

        def round_body(r, carry):
            for s in range(_NBUF):
                j = r * _NBUF + s
                wait(s)
                extract(j, s)
                fire(j + _NBUF, s)
            return carry

        lax.fori_loop(0, n_rounds - 1, round_body, 0)
        for s in range(_NBUF):
            j = (n_rounds - 1) * _NBUF + s
            wait(s)
            extract(jnp.int32(j), s)
        pltpu.sync_copy(embt_v, out_hbm.at[:, pl.ds(base, b_per_w)])

    return k(tablet, idx)


def _mm_body(w_ref, e_ref, b_ref, o_ref):
    o_ref[...] = (
        jnp.dot(w_ref[...], e_ref[...], preferred_element_type=jnp.float32)
        + b_ref[...]
    )


def _linear_tc(embt, W, b2):
    d, bsz = embt.shape
    c = W.shape[0]
    bm = 4096
    return pl.pallas_call(
        _mm_body,
        grid=(bsz // bm,),
        in_specs=[
            pl.BlockSpec((c, d), lambda i: (0, 0)),
            pl.BlockSpec((d, bm), lambda i: (0, i)),
            pl.BlockSpec((c, 1), lambda i: (0, 0)),
        ],
        out_specs=pl.BlockSpec((c, bm), lambda i: (0, i)),
        out_shape=jax.ShapeDtypeStruct((c, bsz), jnp.float32),
    )(W, embt, b2)


def kernel(word_idx, table, W, b):
    embt = _gather_sc(table.T, word_idx.astype(jnp.int32))
    out_t = _linear_tc(embt, W, b.reshape(-1, 1))
    return out_t.T


# speedup vs baseline: 4.0681x; 1.0105x over previous
import functools

import jax
import jax.numpy as jnp
from jax import lax
from jax.experimental import pallas as pl
from jax.experimental.pallas import tpu as pltpu
from jax.experimental.pallas import tpu_sc as plsc

_NBUF = 16


def _gather_sc(tablet, idx):
    nc = 2
    ns = 16
    nw = nc * ns
    b = idx.shape[0]
    d = tablet.shape[0]
    b_per_w = b // nw
    n_rounds = b_per_w // _NBUF
    mesh = plsc.VectorSubcoreMesh(core_axis_name="c", subcore_axis_name="s")

    @functools.partial(
        pl.kernel,
        mesh=mesh,
        out_type=jax.ShapeDtypeStruct((d, b), jnp.float32),
        scratch_types=[
            pltpu.VMEM((b_per_w,), jnp.int32),
            pltpu.SMEM((b_per_w,), jnp.int32),
            pltpu.VMEM((_NBUF, d, 128), jnp.float32),
            pltpu.VMEM((d, b_per_w), jnp.float32),
            pltpu.SemaphoreType.DMA((_NBUF,)),
        ],
        compiler_params=pltpu.CompilerParams(needs_layout_passes=False),
    )
    def k(tablet_hbm, idx_hbm, out_hbm, idx_v, idx_s, blk_v, embt_v, sems):
        wid = lax.axis_index("s") * nc + lax.axis_index("c")
        base = wid * b_per_w
        pltpu.sync_copy(idx_hbm.at[pl.ds(base, b_per_w)], idx_v)
        lanes = lax.iota(jnp.int32, 16)

        def stage(g, carry):
            vec = idx_v[pl.ds(g * 16, 16)]
            for l in range(16):
                idx_s[g * 16 + l] = jnp.sum(jnp.where(lanes == l, vec, 0))
            return carry

        lax.fori_loop(0, b_per_w // 16, stage, 0)

        def fire(j, slot):
            row = idx_s[j]
            cb = pl.multiple_of((row >> 7) * 128, 128)
            for g4 in range(d // 8):
                pltpu.make_async_copy(
                    tablet_hbm.at[pl.ds(g4 * 8, 8), pl.ds(cb, 128)],
                    blk_v.at[slot, pl.ds(g4 * 8, 8)],
                    sems.at[slot],
                ).start()

        def wait(slot):
            pltpu.make_async_copy(
                tablet_hbm.at[:, pl.ds(0, 128)], blk_v.at[slot], sems.at[slot]
            ).wait()

        def extract(j, slot):
            row = idx_s[j]
            lane = jnp.full((16,), row & 127, jnp.int32)
            col = jnp.full((16,), j, jnp.int32)
            for h in range(d // 16):
                part = plsc.load_gather(blk_v.at[slot], [lanes + h * 16, lane])
                plsc.store_scatter(embt_v, [lanes + h * 16, col], part)

        for s in range(_NBUF):
            fire(jnp.int32(s), s)

        def round_body(r, carry):
            for s in range(_NBUF):
                j = r * _NBUF + s
                wait(s)
                extract(j, s)
                fire(j + _NBUF, s)
            return carry

        lax.fori_loop(0, n_rounds - 1, round_body, 0)
        for s in range(_NBUF):
            j = (n_rounds - 1) * _NBUF + s
            wait(s)
            extract(jnp.int32(j), s)
        pltpu.sync_copy(embt_v, out_hbm.at[:, pl.ds(base, b_per_w)])

    return k(tablet, idx)


def _mm_body(w_ref, e_ref, b_ref, o_ref):
    o_ref[...] = (
        jnp.dot(w_ref[...], e_ref[...], preferred_element_type=jnp.float32)
        + b_ref[...]
    )


def _linear_tc(embt, W, b2):
    d, bsz = embt.shape
    c = W.shape[0]
    bm = 8192
    return pl.pallas_call(
        _mm_body,
        grid=(bsz // bm,),
        in_specs=[
            pl.BlockSpec((c, d), lambda i: (0, 0)),
            pl.BlockSpec((d, bm), lambda i: (0, i)),
            pl.BlockSpec((c, 1), lambda i: (0, 0)),
        ],
        out_specs=pl.BlockSpec((c, bm), lambda i: (0, i)),
        out_shape=jax.ShapeDtypeStruct((c, bsz), jnp.float32),
    )(W, embt, b2)


def kernel(word_idx, table, W, b):
    embt = _gather_sc(table.T, word_idx.astype(jnp.int32))
    out_t = _linear_tc(embt, W, b.reshape(-1, 1))
    return out_t.T
